# baseline (speedup 1.0000x reference)
import jax
import jax.numpy as jnp
from jax.experimental import pallas as pl
from jax.experimental.pallas import tpu as pltpu

N_EXP = 16
D_HID = 64
D_OUT = 10


def _moe_block_kernel(xT_ref, Wg_ref, bg_ref, W1_ref, b1_ref, W2_ref, b2_ref,
                      ex_ref, outT_ref):
    xT = xT_ref[...]
    g = jax.lax.dot_general(
        xT, Wg_ref[...], (((0,), (0,)), ((), ()))) + bg_ref[...]
    v1 = jnp.max(g, axis=1, keepdims=True)
    top_cnt = jnp.sum(jnp.where(g == v1, 1.0, 0.0), axis=1, keepdims=True)
    v2 = jnp.max(jnp.where(g < v1, g, -jnp.inf), axis=1, keepdims=True)
    v2 = jnp.where(top_cnt > 1.0, v1, v2)
    z = jnp.where(g >= v2, jnp.exp(g - v1), 0.0)
    coef = z / jnp.sum(z, axis=1, keepdims=True)

    h = jnp.maximum(
        jax.lax.dot_general(xT, W1_ref[...], (((0,), (0,)), ((), ())))
        + b1_ref[...], 0.0)
    ce = jax.lax.dot_general(coef, ex_ref[...], (((1,), (0,)), ((), ())))
    hs = h * ce
    outT = (jax.lax.dot_general(W2_ref[...], hs, (((0,), (1,)), ((), ())))
            + jax.lax.dot_general(b2_ref[...], coef, (((0,), (1,)), ((), ()))))
    outT_ref[...] = outT


@jax.jit
def kernel(x, Wg, bg, W1, b1, W2, b2):
    n_tok, d_in = x.shape
    eh = N_EXP * D_HID
    xT = x.T
    W1all = W1.transpose(1, 0, 2).reshape(d_in, eh)
    b1all = b1.reshape(1, eh)
    W2stack = W2.reshape(eh, D_OUT)
    expand = jnp.repeat(jnp.eye(N_EXP, dtype=jnp.float32), D_HID, axis=1)
    bn = min(2048, n_tok)
    grid = (n_tok // bn,)
    outT = pl.pallas_call(
        _moe_block_kernel,
        grid=grid,
        in_specs=[
            pl.BlockSpec((d_in, bn), lambda i: (0, i)),
            pl.BlockSpec(Wg.shape, lambda i: (0, 0)),
            pl.BlockSpec((1, N_EXP), lambda i: (0, 0)),
            pl.BlockSpec((d_in, eh), lambda i: (0, 0)),
            pl.BlockSpec((1, eh), lambda i: (0, 0)),
            pl.BlockSpec((eh, D_OUT), lambda i: (0, 0)),
            pl.BlockSpec((N_EXP, D_OUT), lambda i: (0, 0)),
            pl.BlockSpec((N_EXP, eh), lambda i: (0, 0)),
        ],
        out_specs=pl.BlockSpec((D_OUT, bn), lambda i: (0, i)),
        out_shape=jax.ShapeDtypeStruct((D_OUT, n_tok), jnp.float32),
        compiler_params=pltpu.CompilerParams(
            dimension_semantics=("arbitrary",)),
    )(xT, Wg, bg.reshape(1, N_EXP), W1all, b1all, W2stack, b2, expand)
    return outT.T


# speedup vs baseline: 1.1749x; 1.1749x over previous
import jax
import jax.numpy as jnp
from jax.experimental import pallas as pl
from jax.experimental.pallas import tpu as pltpu

N_EXP = 16
D_HID = 64
D_OUT = 10


def _moe_block_kernel(xT_ref, Wg_ref, bg_ref, W1_ref, b1_ref, W2_ref, b2_ref,
                      ex_ref, outT_ref):
    xT = xT_ref[...]
    g = jax.lax.dot_general(
        xT, Wg_ref[...], (((0,), (0,)), ((), ()))) + bg_ref[...]
    v1 = jnp.max(g, axis=1, keepdims=True)
    top_cnt = jnp.sum(jnp.where(g == v1, 1.0, 0.0), axis=1, keepdims=True)
    v2 = jnp.max(jnp.where(g < v1, g, -jnp.inf), axis=1, keepdims=True)
    v2 = jnp.where(top_cnt > 1.0, v1, v2)
    z = jnp.where(g >= v2, jnp.exp(g - v1), 0.0)
    coef = z / jnp.sum(z, axis=1, keepdims=True)

    xb = xT.astype(jnp.bfloat16)
    h = jnp.maximum(
        jax.lax.dot_general(xb, W1_ref[...], (((0,), (0,)), ((), ())),
                            preferred_element_type=jnp.float32)
        + b1_ref[...], 0.0)
    ce = jax.lax.dot_general(coef, ex_ref[...], (((1,), (0,)), ((), ())))
    hs = (h * ce).astype(jnp.bfloat16)
    outT = (jax.lax.dot_general(W2_ref[...], hs, (((0,), (1,)), ((), ())),
                                preferred_element_type=jnp.float32)
            + jax.lax.dot_general(b2_ref[...], coef, (((0,), (1,)), ((), ()))))
    outT_ref[...] = outT


@jax.jit
def kernel(x, Wg, bg, W1, b1, W2, b2):
    n_tok, d_in = x.shape
    eh = N_EXP * D_HID
    xT = x.T
    W1all = W1.transpose(1, 0, 2).reshape(d_in, eh).astype(jnp.bfloat16)
    b1all = b1.reshape(1, eh)
    W2stack = W2.reshape(eh, D_OUT).astype(jnp.bfloat16)
    expand = jnp.repeat(jnp.eye(N_EXP, dtype=jnp.float32), D_HID, axis=1)
    bn = min(2048, n_tok)
    grid = (n_tok // bn,)
    outT = pl.pallas_call(
        _moe_block_kernel,
        grid=grid,
        in_specs=[
            pl.BlockSpec((d_in, bn), lambda i: (0, i)),
            pl.BlockSpec(Wg.shape, lambda i: (0, 0)),
            pl.BlockSpec((1, N_EXP), lambda i: (0, 0)),
            pl.BlockSpec((d_in, eh), lambda i: (0, 0)),
            pl.BlockSpec((1, eh), lambda i: (0, 0)),
            pl.BlockSpec((eh, D_OUT), lambda i: (0, 0)),
            pl.BlockSpec((N_EXP, D_OUT), lambda i: (0, 0)),
            pl.BlockSpec((N_EXP, eh), lambda i: (0, 0)),
        ],
        out_specs=pl.BlockSpec((D_OUT, bn), lambda i: (0, i)),
        out_shape=jax.ShapeDtypeStruct((D_OUT, n_tok), jnp.float32),
        compiler_params=pltpu.CompilerParams(
            dimension_semantics=("arbitrary",)),
    )(xT, Wg, bg.reshape(1, N_EXP), W1all, b1all, W2stack, b2, expand)
    return outT.T
